# baseline (speedup 1.0000x reference)
import functools
import jax
import jax.numpy as jnp
from jax import lax
from jax.experimental import pallas as pl
from jax.experimental.pallas import tpu as pltpu
from jax.experimental.pallas import tpu_sc as plsc

B, N, D, K = 8, 1024, 128, 16
E = B * N * K
EPS = 1e-5

TE = 16384
TN = TE // K
GE = E // TE

F32 = jnp.float32
BF16 = jnp.bfloat16


def _knn_body(pts_ref, out_ref, p_ref):
    x = pts_ref[0]
    xx = jnp.sum(x * x, axis=0)
    xb = x.astype(BF16).astype(F32)
    g = (xb[0][:, None] * xb[0][None, :]
         + xb[1][:, None] * xb[1][None, :]
         + xb[2][:, None] * xb[2][None, :])
    inner = -2.0 * g
    col = lax.broadcasted_iota(jnp.int32, (N, N), 1)
    p = (-xx[:, None] - inner) - xx[None, :]
    p_ref[...] = p
    m = jnp.max(p, axis=1, keepdims=True)
    b = pl.program_id(0)
    rows = []
    for t in range(K + 1):
        pv = p_ref[...]
        am = jnp.min(jnp.where(pv == m, col, N), axis=1)
        if t > 0:
            rows.append(am + b * N)
        if t < K:
            m = jnp.max(jnp.where(pv < m, pv, -jnp.inf), axis=1, keepdims=True)
    out_ref[0] = jnp.stack(rows, axis=0).astype(jnp.int32)


def _knn(points):
    return pl.pallas_call(
        _knn_body,
        grid=(B,),
        in_specs=[pl.BlockSpec((1, 3, N), lambda b: (b, 0, 0))],
        out_specs=pl.BlockSpec((1, K, N), lambda b: (b, 0, 0)),
        out_shape=jax.ShapeDtypeStruct((B, K, N), jnp.int32),
        scratch_shapes=[pltpu.VMEM((N, N), F32)],
    )(points)


_NW = 32
_PW = E // _NW
_CH = 512
_NCH = _PW // _CH


def _make_gather():
    mesh = plsc.VectorSubcoreMesh(core_axis_name="c", subcore_axis_name="s")

    @functools.partial(
        pl.kernel, mesh=mesh,
        compiler_params=pltpu.CompilerParams(use_tc_tiling_on_sc=False),
        out_type=jax.ShapeDtypeStruct((E, D // 2), jnp.int32),
        scratch_types=[
            pltpu.VMEM((_PW,), jnp.int32),
            pltpu.VMEM((_CH, D // 2), jnp.int32),
            pltpu.VMEM((_CH, D // 2), jnp.int32),
            pltpu.VMEM((_CH, D // 2), jnp.int32),
            pltpu.SemaphoreType.DMA,
            pltpu.SemaphoreType.DMA,
            pltpu.SemaphoreType.DMA,
            pltpu.SemaphoreType.DMA,
            pltpu.SemaphoreType.DMA,
            pltpu.SemaphoreType.DMA,
        ],
    )
    def gather(table_hbm, idx_hbm, out_hbm, idx_v, b0, b1, b2,
               gs0, gs1, gs2, ws0, ws1, ws2):
        bufs = (b0, b1, b2)
        gs = (gs0, gs1, gs2)
        ws = (ws0, ws1, ws2)
        wid = lax.axis_index("s") * 2 + lax.axis_index("c")
        base = wid * _PW
        pltpu.sync_copy(idx_hbm.at[pl.ds(base, _PW)], idx_v)
        gd = [None] * 3
        wd = [None] * 3
        for c in range(_NCH):
            k = c % 3
            if wd[k] is not None:
                wd[k].wait()
            gd[k] = pltpu.async_copy(
                table_hbm.at[idx_v.at[pl.ds(c * _CH, _CH)]], bufs[k], gs[k])
            if c >= 2:
                j = (c - 2) % 3
                gd[j].wait()
                wd[j] = pltpu.async_copy(
                    bufs[j], out_hbm.at[pl.ds(base + (c - 2) * _CH, _CH)], ws[j])
        for c in (_NCH - 2, _NCH - 1):
            j = c % 3
            gd[j].wait()
            wd[j] = pltpu.async_copy(
                bufs[j], out_hbm.at[pl.ds(base + c * _CH, _CH)], ws[j])
        for j in range(3):
            if wd[j] is not None:
                wd[j].wait()

    return gather


_gather_sc = _make_gather()


def _acc_init(refs):
    @pl.when(pl.program_id(0) == 0)
    def _():
        for r in refs:
            r[...] = jnp.zeros_like(r)


def _conv0_body(nbr_ref, ft_ref, w_ref, a_ref, y_ref, s_ref, ss_ref):
    c = jnp.dot(ft_ref[...], a_ref[...], preferred_element_type=F32)
    crep = jnp.broadcast_to(c[:, None, :], (TN, K, c.shape[1])).reshape(TE, -1)
    xi = nbr_ref[...]
    lo = lax.bitcast_convert_type(xi << 16, F32).astype(BF16)
    hi = lax.bitcast_convert_type(xi & jnp.int32(-65536), F32).astype(BF16)
    y = (jnp.dot(lo, w_ref[: D // 2], preferred_element_type=F32)
         + jnp.dot(hi, w_ref[D // 2:], preferred_element_type=F32) + crep)
    y_ref[...] = y.astype(BF16)
    _acc_init((s_ref, ss_ref))
    s_ref[...] += jnp.sum(y, axis=0, keepdims=True)
    ss_ref[...] += jnp.sum(y * y, axis=0, keepdims=True)


def _conv0(nbr, ft, w0bT, aT):
    return pl.pallas_call(
        _conv0_body,
        grid=(GE,),
        in_specs=[
            pl.BlockSpec((TE, D // 2), lambda i: (i, 0)),
            pl.BlockSpec((TN, D), lambda i: (i, 0)),
            pl.BlockSpec((D, D), lambda i: (0, 0)),
            pl.BlockSpec((D, D), lambda i: (0, 0)),
        ],
        out_specs=[
            pl.BlockSpec((TE, D), lambda i: (i, 0)),
            pl.BlockSpec((1, D), lambda i: (0, 0)),
            pl.BlockSpec((1, D), lambda i: (0, 0)),
        ],
        out_shape=[
            jax.ShapeDtypeStruct((E, D), BF16),
            jax.ShapeDtypeStruct((1, D), F32),
            jax.ShapeDtypeStruct((1, D), F32),
        ],
    )(nbr, ft, w0bT, aT)


def _stats1_body(y0_ref, sc_ref, sh_ref, ones_ref, g_ref, sz_ref):
    z = jnp.maximum(y0_ref[...] * sc_ref[...] + sh_ref[...], BF16(0.0))
    _acc_init((g_ref, sz_ref))
    g_ref[...] += lax.dot_general(z, z, (((0,), (0,)), ((), ())),
                                  preferred_element_type=F32)
    sz_ref[...] += jnp.dot(ones_ref[...], z, preferred_element_type=F32)


def _stats1(y0, scale0, shift0):
    ones = jnp.ones((1, TE), BF16)
    return pl.pallas_call(
        _stats1_body,
        compiler_params=pltpu.CompilerParams(fuse_transposed_lhs_in_matmul=True),
        grid=(GE,),
        in_specs=[
            pl.BlockSpec((TE, D), lambda i: (i, 0)),
            pl.BlockSpec((1, D), lambda i: (0, 0)),
            pl.BlockSpec((1, D), lambda i: (0, 0)),
            pl.BlockSpec((1, TE), lambda i: (0, 0)),
        ],
        out_specs=[
            pl.BlockSpec((D, D), lambda i: (0, 0)),
            pl.BlockSpec((1, D), lambda i: (0, 0)),
        ],
        out_shape=[
            jax.ShapeDtypeStruct((D, D), F32),
            jax.ShapeDtypeStruct((1, D), F32),
        ],
    )(y0, scale0, shift0, ones)


def _stats2_body(y0_ref, ft_ref, sc0_ref, sh0_ref, sc1_ref, sh1_ref,
                 w1_ref, wscT_ref, ones_ref,
                 g_ref, sz_ref, scut_ref, ssc_ref, sssc_ref):
    z0 = jnp.maximum(y0_ref[...] * sc0_ref[...] + sh0_ref[...], BF16(0.0))
    y1 = jnp.dot(z0, w1_ref[...], preferred_element_type=F32)
    z1 = jnp.maximum(y1 * sc1_ref[...] + sh1_ref[...], 0.0).astype(BF16)
    scut = jnp.dot(ft_ref[...], wscT_ref[...], preferred_element_type=F32)
    scut_ref[...] = scut
    _acc_init((g_ref, sz_ref, ssc_ref, sssc_ref))
    g_ref[...] += lax.dot_general(z1, z1, (((0,), (0,)), ((), ())),
                                  preferred_element_type=F32)
    sz_ref[...] += jnp.dot(ones_ref[...], z1, preferred_element_type=F32)
    ssc_ref[...] += jnp.sum(scut, axis=0, keepdims=True)
    sssc_ref[...] += jnp.sum(scut * scut, axis=0, keepdims=True)


def _stats2(y0, ft, scale0, shift0, scale1, shift1, w1T, wscT):
    C2 = wscT.shape[1]
    ones = jnp.ones((1, TE), BF16)
    return pl.pallas_call(
        _stats2_body,
        compiler_params=pltpu.CompilerParams(fuse_transposed_lhs_in_matmul=True),
        grid=(GE,),
        in_specs=[
            pl.BlockSpec((TE, D), lambda i: (i, 0)),
            pl.BlockSpec((TN, D), lambda i: (i, 0)),
            pl.BlockSpec((1, D), lambda i: (0, 0)),
            pl.BlockSpec((1, D), lambda i: (0, 0)),
            pl.BlockSpec((1, D), lambda i: (0, 0)),
            pl.BlockSpec((1, D), lambda i: (0, 0)),
            pl.BlockSpec((D, D), lambda i: (0, 0)),
            pl.BlockSpec((D, C2), lambda i: (0, 0)),
            pl.BlockSpec((1, TE), lambda i: (0, 0)),
        ],
        out_specs=[
            pl.BlockSpec((D, D), lambda i: (0, 0)),
            pl.BlockSpec((1, D), lambda i: (0, 0)),
            pl.BlockSpec((TN, C2), lambda i: (i, 0)),
            pl.BlockSpec((1, C2), lambda i: (0, 0)),
            pl.BlockSpec((1, C2), lambda i: (0, 0)),
        ],
        out_shape=[
            jax.ShapeDtypeStruct((D, D), F32),
            jax.ShapeDtypeStruct((1, D), F32),
            jax.ShapeDtypeStruct((B * N, C2), F32),
            jax.ShapeDtypeStruct((1, C2), F32),
            jax.ShapeDtypeStruct((1, C2), F32),
        ],
    )(y0, ft, scale0, shift0, scale1, shift1, w1T, wscT, ones)


def _final_body(y0_ref, scut_ref, sc0_ref, sh0_ref, sc1_ref, sh1_ref,
                w1_ref, waug_ref, scsc_ref, shsc_ref, out_ref):
    C2 = waug_ref.shape[1]
    z0 = jnp.maximum(y0_ref[0] * sc0_ref[...] + sh0_ref[...], BF16(0.0))
    y1 = jnp.dot(z0, w1_ref[...], preferred_element_type=F32)
    z1 = jnp.maximum(y1 * sc1_ref[...] + sh1_ref[...], 0.0).astype(BF16)
    ones = jnp.ones((TE, 1), BF16)
    z1a = jnp.concatenate([z1, ones], axis=1)
    z2 = jnp.maximum(jnp.dot(z1a, waug_ref[...],
                             preferred_element_type=F32), 0.0)
    fts = jnp.mean(z2.reshape(TN, K, C2), axis=1)
    o = jnp.maximum(scut_ref[0] * scsc_ref[...] + shsc_ref[...] + fts, 0.0)
    out_ref[0] = o.T


def _final(y0, scut, scale0, shift0, scale1, shift1, w1T, waug,
           scale_sc, shift_sc):
    C2 = waug.shape[1]
    y0v = y0.reshape(B, N * K, D)
    scv = scut.reshape(B, N, C2)
    GN = N // TN
    return pl.pallas_call(
        _final_body,
        grid=(B, GN),
        in_specs=[
            pl.BlockSpec((1, TE, D), lambda b, j: (b, j, 0)),
            pl.BlockSpec((1, TN, C2), lambda b, j: (b, j, 0)),
            pl.BlockSpec((1, D), lambda b, j: (0, 0)),
            pl.BlockSpec((1, D), lambda b, j: (0, 0)),
            pl.BlockSpec((1, D), lambda b, j: (0, 0)),
            pl.BlockSpec((1, D), lambda b, j: (0, 0)),
            pl.BlockSpec((D, D), lambda b, j: (0, 0)),
            pl.BlockSpec((D + 1, C2), lambda b, j: (0, 0)),
            pl.BlockSpec((1, C2), lambda b, j: (0, 0)),
            pl.BlockSpec((1, C2), lambda b, j: (0, 0)),
        ],
        out_specs=pl.BlockSpec((1, C2, TN), lambda b, j: (b, 0, j)),
        out_shape=jax.ShapeDtypeStruct((B, C2, N), F32),
    )(y0v, scv, scale0, shift0, scale1, shift1, w1T, waug,
      scale_sc, shift_sc)


def _fold(count, s, ss, g, bb):
    m = s[0] / count
    v = ss[0] / count - m * m
    scale = g / jnp.sqrt(v + EPS)
    shift = bb - m * scale
    return scale.reshape(1, -1), shift.reshape(1, -1)


def _fold_gram(count, sz, G, W, g, bb):
    s = sz[0] @ W
    ss = jnp.sum(W * (G @ W), axis=0)
    m = s / count
    v = ss / count - m * m
    scale = g / jnp.sqrt(v + EPS)
    shift = bb - m * scale
    return scale.reshape(1, -1), shift.reshape(1, -1)


def kernel(points, features, W0, g0, b0, W1, g1, b1, W2, g2, b2, Wsc, gsc, bsc):
    gidx = _knn(points)
    idx_flat = jnp.transpose(gidx, (0, 2, 1)).reshape(-1)
    ftb = jnp.swapaxes(features, 1, 2).reshape(B * N, D).astype(BF16)
    u = lax.bitcast_convert_type(ftb, jnp.uint16).astype(jnp.uint32)
    packed = lax.bitcast_convert_type(u[:, :D // 2] | (u[:, D // 2:] << 16),
                                      jnp.int32)

    nbr = _gather_sc(packed, idx_flat)

    W0a, W0b = W0[:, :D], W0[:, D:]
    aT = (W0a - W0b).T.astype(BF16)
    w0bT = W0b.T.astype(BF16)
    w1T = W1.T.astype(BF16)
    wscT = Wsc.T.astype(BF16)

    y0, s0, ss0 = _conv0(nbr, ftb, w0bT, aT)
    sc0, sh0 = _fold(E, s0, ss0, g0, b0)
    sc0b, sh0b = sc0.astype(BF16), sh0.astype(BF16)
    G0, sz0 = _stats1(y0, sc0b, sh0b)
    sc1, sh1 = _fold_gram(E, sz0, G0, W1.T, g1, b1)
    G1, sz1, scut, ssc, sssc = _stats2(y0, ftb, sc0b, sh0b, sc1, sh1,
                                       w1T, wscT)
    sc2, sh2 = _fold_gram(E, sz1, G1, W2.T, g2, b2)
    scsc, shsc = _fold(B * N, ssc, sssc, gsc, bsc)
    waug = jnp.concatenate([W2.T * sc2, sh2], axis=0).astype(BF16)
    return _final(y0, scut, sc0b, sh0b, sc1, sh1, w1T, waug, scsc, shsc)


# speedup vs baseline: 1.0508x; 1.0508x over previous
import functools
import jax
import jax.numpy as jnp
from jax import lax
from jax.experimental import pallas as pl
from jax.experimental.pallas import tpu as pltpu
from jax.experimental.pallas import tpu_sc as plsc

B, N, D, K = 8, 1024, 128, 16
E = B * N * K
EPS = 1e-5

TE = 16384
TN = TE // K
GE = E // TE

F32 = jnp.float32
BF16 = jnp.bfloat16


def _knn_body(pts_ref, out_ref, p_ref):
    x = pts_ref[0]
    xx = jnp.sum(x * x, axis=0)
    xb = x.astype(BF16).astype(F32)
    g = (xb[0][:, None] * xb[0][None, :]
         + xb[1][:, None] * xb[1][None, :]
         + xb[2][:, None] * xb[2][None, :])
    inner = -2.0 * g
    col = lax.broadcasted_iota(jnp.int32, (N, N), 1)
    p = (-xx[:, None] - inner) - xx[None, :]
    p_ref[...] = p
    m = jnp.max(p, axis=1, keepdims=True)
    b = pl.program_id(0)
    rows = []
    for t in range(K + 1):
        pv = p_ref[...]
        am = jnp.min(jnp.where(pv == m, col, N), axis=1)
        if t > 0:
            rows.append(am + b * N)
        if t < K:
            m = jnp.max(jnp.where(pv < m, pv, -jnp.inf), axis=1, keepdims=True)
    out_ref[0] = jnp.stack(rows, axis=0).astype(jnp.int32)


def _knn(points):
    return pl.pallas_call(
        _knn_body,
        grid=(B,),
        in_specs=[pl.BlockSpec((1, 3, N), lambda b: (b, 0, 0))],
        out_specs=pl.BlockSpec((1, K, N), lambda b: (b, 0, 0)),
        out_shape=jax.ShapeDtypeStruct((B, K, N), jnp.int32),
        scratch_shapes=[pltpu.VMEM((N, N), F32)],
    )(points)


_NW = 32
_PW = E // _NW
_CH = 512
_NCH = _PW // _CH


def _make_gather():
    mesh = plsc.VectorSubcoreMesh(core_axis_name="c", subcore_axis_name="s")

    @functools.partial(
        pl.kernel, mesh=mesh,
        compiler_params=pltpu.CompilerParams(use_tc_tiling_on_sc=False),
        out_type=jax.ShapeDtypeStruct((E, D // 2), jnp.int32),
        scratch_types=[
            pltpu.VMEM((_PW,), jnp.int32),
            pltpu.VMEM((_CH, D // 2), jnp.int32),
            pltpu.VMEM((_CH, D // 2), jnp.int32),
            pltpu.VMEM((_CH, D // 2), jnp.int32),
            pltpu.SemaphoreType.DMA,
            pltpu.SemaphoreType.DMA,
            pltpu.SemaphoreType.DMA,
            pltpu.SemaphoreType.DMA,
            pltpu.SemaphoreType.DMA,
            pltpu.SemaphoreType.DMA,
        ],
    )
    def gather(table_hbm, idx_hbm, out_hbm, idx_v, b0, b1, b2,
               gs0, gs1, gs2, ws0, ws1, ws2):
        bufs = (b0, b1, b2)
        gs = (gs0, gs1, gs2)
        ws = (ws0, ws1, ws2)
        wid = lax.axis_index("s") * 2 + lax.axis_index("c")
        base = wid * _PW
        pltpu.sync_copy(idx_hbm.at[pl.ds(base, _PW)], idx_v)
        gd = [None] * 3
        wd = [None] * 3
        for c in range(_NCH):
            k = c % 3
            if wd[k] is not None:
                wd[k].wait()
            gd[k] = pltpu.async_copy(
                table_hbm.at[idx_v.at[pl.ds(c * _CH, _CH)]], bufs[k], gs[k])
            if c >= 2:
                j = (c - 2) % 3
                gd[j].wait()
                wd[j] = pltpu.async_copy(
                    bufs[j], out_hbm.at[pl.ds(base + (c - 2) * _CH, _CH)], ws[j])
        for c in (_NCH - 2, _NCH - 1):
            j = c % 3
            gd[j].wait()
            wd[j] = pltpu.async_copy(
                bufs[j], out_hbm.at[pl.ds(base + c * _CH, _CH)], ws[j])
        for j in range(3):
            if wd[j] is not None:
                wd[j].wait()

    return gather


_gather_sc = _make_gather()


def _acc_init(refs):
    @pl.when(pl.program_id(0) == 0)
    def _():
        for r in refs:
            r[...] = jnp.zeros_like(r)


def _conv0_body(nbr_ref, ft_ref, w_ref, a_ref, y_ref, s_ref, ss_ref):
    c = jnp.dot(ft_ref[...], a_ref[...], preferred_element_type=F32)
    crep = jnp.broadcast_to(c[:, None, :], (TN, K, c.shape[1])).reshape(TE, -1)
    xi = nbr_ref[...]
    lo = lax.bitcast_convert_type(xi << 16, F32)
    hi = lax.bitcast_convert_type(xi & jnp.int32(-65536), F32)
    nbr = jnp.concatenate([lo, hi], axis=1).astype(BF16)
    y = jnp.dot(nbr, w_ref[...], preferred_element_type=F32) + crep
    y_ref[...] = y.astype(BF16)
    _acc_init((s_ref, ss_ref))
    s_ref[...] += jnp.sum(y, axis=0, keepdims=True)
    ss_ref[...] += jnp.sum(y * y, axis=0, keepdims=True)


def _conv0(nbr, ft, w0bT, aT):
    return pl.pallas_call(
        _conv0_body,
        grid=(GE,),
        in_specs=[
            pl.BlockSpec((TE, D // 2), lambda i: (i, 0)),
            pl.BlockSpec((TN, D), lambda i: (i, 0)),
            pl.BlockSpec((D, D), lambda i: (0, 0)),
            pl.BlockSpec((D, D), lambda i: (0, 0)),
        ],
        out_specs=[
            pl.BlockSpec((TE, D), lambda i: (i, 0)),
            pl.BlockSpec((1, D), lambda i: (0, 0)),
            pl.BlockSpec((1, D), lambda i: (0, 0)),
        ],
        out_shape=[
            jax.ShapeDtypeStruct((E, D), BF16),
            jax.ShapeDtypeStruct((1, D), F32),
            jax.ShapeDtypeStruct((1, D), F32),
        ],
    )(nbr, ft, w0bT, aT)


def _stats1_body(y0_ref, sc_ref, sh_ref, ones_ref, g_ref, sz_ref):
    z = jnp.maximum(y0_ref[...] * sc_ref[...] + sh_ref[...], BF16(0.0))
    _acc_init((g_ref, sz_ref))
    g_ref[...] += lax.dot_general(z, z, (((0,), (0,)), ((), ())),
                                  preferred_element_type=F32)
    sz_ref[...] += jnp.dot(ones_ref[...], z, preferred_element_type=F32)


def _stats1(y0, scale0, shift0):
    ones = jnp.ones((1, TE), BF16)
    return pl.pallas_call(
        _stats1_body,
        grid=(GE,),
        in_specs=[
            pl.BlockSpec((TE, D), lambda i: (i, 0)),
            pl.BlockSpec((1, D), lambda i: (0, 0)),
            pl.BlockSpec((1, D), lambda i: (0, 0)),
            pl.BlockSpec((1, TE), lambda i: (0, 0)),
        ],
        out_specs=[
            pl.BlockSpec((D, D), lambda i: (0, 0)),
            pl.BlockSpec((1, D), lambda i: (0, 0)),
        ],
        out_shape=[
            jax.ShapeDtypeStruct((D, D), F32),
            jax.ShapeDtypeStruct((1, D), F32),
        ],
    )(y0, scale0, shift0, ones)


def _stats2_body(y0_ref, ft_ref, sc0_ref, sh0_ref, sc1_ref, sh1_ref,
                 w1_ref, wscT_ref, ones_ref,
                 g_ref, sz_ref, scut_ref, ssc_ref, sssc_ref):
    z0 = jnp.maximum(y0_ref[...] * sc0_ref[...] + sh0_ref[...], BF16(0.0))
    y1 = jnp.dot(z0, w1_ref[...], preferred_element_type=F32)
    z1 = jnp.maximum(y1 * sc1_ref[...] + sh1_ref[...], 0.0).astype(BF16)
    scut = jnp.dot(ft_ref[...], wscT_ref[...], preferred_element_type=F32)
    scut_ref[...] = scut
    _acc_init((g_ref, sz_ref, ssc_ref, sssc_ref))
    g_ref[...] += lax.dot_general(z1, z1, (((0,), (0,)), ((), ())),
                                  preferred_element_type=F32)
    sz_ref[...] += jnp.dot(ones_ref[...], z1, preferred_element_type=F32)
    ssc_ref[...] += jnp.sum(scut, axis=0, keepdims=True)
    sssc_ref[...] += jnp.sum(scut * scut, axis=0, keepdims=True)


def _stats2(y0, ft, scale0, shift0, scale1, shift1, w1T, wscT):
    C2 = wscT.shape[1]
    ones = jnp.ones((1, TE), BF16)
    return pl.pallas_call(
        _stats2_body,
        grid=(GE,),
        in_specs=[
            pl.BlockSpec((TE, D), lambda i: (i, 0)),
            pl.BlockSpec((TN, D), lambda i: (i, 0)),
            pl.BlockSpec((1, D), lambda i: (0, 0)),
            pl.BlockSpec((1, D), lambda i: (0, 0)),
            pl.BlockSpec((1, D), lambda i: (0, 0)),
            pl.BlockSpec((1, D), lambda i: (0, 0)),
            pl.BlockSpec((D, D), lambda i: (0, 0)),
            pl.BlockSpec((D, C2), lambda i: (0, 0)),
            pl.BlockSpec((1, TE), lambda i: (0, 0)),
        ],
        out_specs=[
            pl.BlockSpec((D, D), lambda i: (0, 0)),
            pl.BlockSpec((1, D), lambda i: (0, 0)),
            pl.BlockSpec((TN, C2), lambda i: (i, 0)),
            pl.BlockSpec((1, C2), lambda i: (0, 0)),
            pl.BlockSpec((1, C2), lambda i: (0, 0)),
        ],
        out_shape=[
            jax.ShapeDtypeStruct((D, D), F32),
            jax.ShapeDtypeStruct((1, D), F32),
            jax.ShapeDtypeStruct((B * N, C2), F32),
            jax.ShapeDtypeStruct((1, C2), F32),
            jax.ShapeDtypeStruct((1, C2), F32),
        ],
    )(y0, ft, scale0, shift0, scale1, shift1, w1T, wscT, ones)


def _final_body(y0_ref, scut_ref, sc0_ref, sh0_ref, sc1_ref, sh1_ref,
                w1_ref, waug_ref, scsc_ref, shsc_ref, out_ref):
    C2 = waug_ref.shape[1]
    z0 = jnp.maximum(y0_ref[0] * sc0_ref[...] + sh0_ref[...], BF16(0.0))
    y1 = jnp.dot(z0, w1_ref[...], preferred_element_type=F32)
    z1 = jnp.maximum(y1 * sc1_ref[...] + sh1_ref[...], 0.0).astype(BF16)
    ones = jnp.ones((TE, 1), BF16)
    z1a = jnp.concatenate([z1, ones], axis=1)
    z2 = jnp.maximum(jnp.dot(z1a, waug_ref[...],
                             preferred_element_type=F32), 0.0)
    fts = jnp.mean(z2.reshape(TN, K, C2), axis=1)
    o = jnp.maximum(scut_ref[0] * scsc_ref[...] + shsc_ref[...] + fts, 0.0)
    out_ref[0] = o.T


def _final(y0, scut, scale0, shift0, scale1, shift1, w1T, waug,
           scale_sc, shift_sc):
    C2 = waug.shape[1]
    y0v = y0.reshape(B, N * K, D)
    scv = scut.reshape(B, N, C2)
    GN = N // TN
    return pl.pallas_call(
        _final_body,
        grid=(B, GN),
        in_specs=[
            pl.BlockSpec((1, TE, D), lambda b, j: (b, j, 0)),
            pl.BlockSpec((1, TN, C2), lambda b, j: (b, j, 0)),
            pl.BlockSpec((1, D), lambda b, j: (0, 0)),
            pl.BlockSpec((1, D), lambda b, j: (0, 0)),
            pl.BlockSpec((1, D), lambda b, j: (0, 0)),
            pl.BlockSpec((1, D), lambda b, j: (0, 0)),
            pl.BlockSpec((D, D), lambda b, j: (0, 0)),
            pl.BlockSpec((D + 1, C2), lambda b, j: (0, 0)),
            pl.BlockSpec((1, C2), lambda b, j: (0, 0)),
            pl.BlockSpec((1, C2), lambda b, j: (0, 0)),
        ],
        out_specs=pl.BlockSpec((1, C2, TN), lambda b, j: (b, 0, j)),
        out_shape=jax.ShapeDtypeStruct((B, C2, N), F32),
    )(y0v, scv, scale0, shift0, scale1, shift1, w1T, waug,
      scale_sc, shift_sc)


def _fold(count, s, ss, g, bb):
    m = s[0] / count
    v = ss[0] / count - m * m
    scale = g / jnp.sqrt(v + EPS)
    shift = bb - m * scale
    return scale.reshape(1, -1), shift.reshape(1, -1)


def _fold_gram(count, sz, G, W, g, bb):
    s = sz[0] @ W
    ss = jnp.sum(W * (G @ W), axis=0)
    m = s / count
    v = ss / count - m * m
    scale = g / jnp.sqrt(v + EPS)
    shift = bb - m * scale
    return scale.reshape(1, -1), shift.reshape(1, -1)


def kernel(points, features, W0, g0, b0, W1, g1, b1, W2, g2, b2, Wsc, gsc, bsc):
    gidx = _knn(points)
    idx_flat = jnp.transpose(gidx, (0, 2, 1)).reshape(-1)
    ftb = jnp.swapaxes(features, 1, 2).reshape(B * N, D).astype(BF16)
    u = lax.bitcast_convert_type(ftb, jnp.uint16).astype(jnp.uint32)
    packed = lax.bitcast_convert_type(u[:, :D // 2] | (u[:, D // 2:] << 16),
                                      jnp.int32)

    nbr = _gather_sc(packed, idx_flat)

    W0a, W0b = W0[:, :D], W0[:, D:]
    aT = (W0a - W0b).T.astype(BF16)
    w0bT = W0b.T.astype(BF16)
    w1T = W1.T.astype(BF16)
    wscT = Wsc.T.astype(BF16)

    y0, s0, ss0 = _conv0(nbr, ftb, w0bT, aT)
    sc0, sh0 = _fold(E, s0, ss0, g0, b0)
    sc0b, sh0b = sc0.astype(BF16), sh0.astype(BF16)
    G0, sz0 = _stats1(y0, sc0b, sh0b)
    sc1, sh1 = _fold_gram(E, sz0, G0, W1.T, g1, b1)
    G1, sz1, scut, ssc, sssc = _stats2(y0, ftb, sc0b, sh0b, sc1, sh1,
                                       w1T, wscT)
    sc2, sh2 = _fold_gram(E, sz1, G1, W2.T, g2, b2)
    scsc, shsc = _fold(B * N, ssc, sssc, gsc, bsc)
    waug = jnp.concatenate([W2.T * sc2, sh2], axis=0).astype(BF16)
    return _final(y0, scut, sc0b, sh0b, sc1, sh1, w1T, waug, scsc, shsc)
